# baseline (speedup 1.0000x reference)
import jax
import jax.numpy as jnp
from jax import lax
from jax.experimental import pallas as pl
from jax.experimental.pallas import tpu as pltpu
from jax.experimental.pallas import tpu_sc as plsc

N_NODES = 50000
E_EDGES = 800000
H = 64
HH = 32
R = 4
NB_BASES = 30

CHUNK = 128
ROWS = 6400
E_PAD = ROWS * CHUNK
RN = R * N_NODES
RN_PAD = 200704
NPAD = 51200
KB = 40
NT = 16
SB = 8
SB1 = 2


def _w_kernel(comp1_ref, b1flat_ref, b2ct_ref, comp2t_ref, w1_ref, w2_ref):
    w1_ref[...] = jnp.dot(comp1_ref[...], b1flat_ref[...],
                          preferred_element_type=jnp.float32)
    w2_ref[...] = jnp.dot(b2ct_ref[...], comp2t_ref[...],
                          preferred_element_type=jnp.float32)


BN1 = 2000


def _dense1_kernel(x_ref, w1_ref, root_ref, bias_ref, tab_ref, rp_ref):
    xb = x_ref[...]
    for r in range(R):
        tab_ref[0, r] = jnp.dot(xb, w1_ref[0, r],
                                preferred_element_type=jnp.float32)
    rp_ref[0] = jnp.dot(xb, root_ref[0],
                        preferred_element_type=jnp.float32) + bias_ref[0]


def _invc_kernel(parts_ref, invc_ref):
    p = parts_ref[0] + parts_ref[1]
    inv = 1.0 / jnp.maximum(p, 1.0)
    nrows = RN_PAD // 128
    g = (lax.broadcasted_iota(jnp.int32, (nrows, 128), 0) * 128
         + lax.broadcasted_iota(jnp.int32, (nrows, 128), 1))
    invc_ref[...] = jnp.where(g < RN, inv, 0.0)


def _dense2_kernel(a0_ref, a1_ref, wa_ref, wb_ref, bias_ref, out_ref):
    out_ref[...] = (
        jnp.dot(a0_ref[0], wa_ref[...], preferred_element_type=jnp.float32)
        + jnp.dot(a1_ref[0], wb_ref[...], preferred_element_type=jnp.float32)
        + bias_ref[...])


def _fin_kernel(a_ref, o_ref):
    o_ref[...] = a_ref[0] + a_ref[1]


_MESH = plsc.VectorSubcoreMesh(core_axis_name="c", subcore_axis_name="s")


def _counts_body(cidx_hbm, out_hbm, acc_sh, cbuf, ones, zbuf, sem):
    c = lax.axis_index("c")
    s = lax.axis_index("s")
    nz = RN_PAD // NT

    def zf(i, _):
        zbuf[pl.ds(i * 16, 16)] = jnp.zeros((16,), jnp.float32)
        return 0
    lax.fori_loop(0, nz // 16, zf, 0)
    pltpu.sync_copy(zbuf, acc_sh.at[pl.ds(s * nz, nz)])

    def of(i, _):
        ones[pl.ds(i * 16, 16)] = jnp.ones((16,), jnp.float32)
        return 0
    lax.fori_loop(0, CHUNK // 16, of, 0)
    plsc.subcore_barrier()

    base = (c * NT + s) * (ROWS // (2 * NT))

    def blk(b, _):
        pltpu.sync_copy(cidx_hbm.at[pl.ds(base + b * KB, KB)], cbuf)

        def ch(j, _):
            pltpu.sync_copy(ones, acc_sh.at[cbuf.at[j]], add=True)
            return 0
        lax.fori_loop(0, KB, ch, 0)
        return 0
    lax.fori_loop(0, (ROWS // (2 * NT)) // KB, blk, 0)

    plsc.subcore_barrier()
    pltpu.sync_copy(acc_sh.at[pl.ds(s * nz, nz)], zbuf)
    pltpu.sync_copy(zbuf, out_hbm.at[c, pl.ds(s * nz, nz)])


def _l1_body(gidx_hbm, cidx_hbm, dst_hbm, tab_hbm, invc_hbm, rootp_hbm,
             out_hbm, sce_hbm, acc_sh, gA, cA, dA, gB, cB, dB, rowsA, rowsB,
             cvA, cvB, semgA, semgB, semsA, semsB):
    c = lax.axis_index("c")
    s = lax.axis_index("s")
    nr = NPAD // NT
    BR = 200

    def ini(i, _):
        o = s * nr + i * BR
        pltpu.sync_copy(rootp_hbm.at[c, pl.ds(o, BR)],
                        rowsA.at[pl.ds(0, BR)])
        pltpu.sync_copy(rowsA.at[pl.ds(0, BR)], acc_sh.at[pl.ds(o, BR)])
        return 0
    lax.fori_loop(0, nr // BR, ini, 0)
    plsc.subcore_barrier()

    def fire(o, g, cc, dd, rows, cv, semg):
        pltpu.sync_copy(gidx_hbm.at[pl.ds(o, SB1)], g)
        pltpu.sync_copy(cidx_hbm.at[pl.ds(o, SB1)], cc)
        pltpu.sync_copy(dst_hbm.at[pl.ds(o, SB1)], dd)
        ds_ = []
        for k in range(SB1):
            ds_.append(pltpu.async_copy(
                tab_hbm.at[c].at[g.at[k]],
                rows.at[pl.ds(k * CHUNK, CHUNK)], semg))
            ds_.append(pltpu.async_copy(
                invc_hbm.at[cc.at[k]],
                cv.at[k], semg))
        return ds_

    def scale(rows, cv):
        def sck(k, _):
            sv16 = cv[k // 8, pl.ds((k % 8) * 16, 16)]
            for e16 in range(16):
                e = k * 16 + e16
                sv = sv16[e16]
                rows[e, pl.ds(0, 16)] = rows[e, pl.ds(0, 16)] * sv
                rows[e, pl.ds(16, 16)] = rows[e, pl.ds(16, 16)] * sv
            return 0
        lax.fori_loop(0, SB1 * CHUNK // 16, sck, 0)

    def scat(rows, dd, sems):
        ds_ = []
        for k in range(SB1):
            ds_.append(pltpu.async_copy(
                rows.at[pl.ds(k * CHUNK, CHUNK)],
                acc_sh.at[dd.at[k]], sems, add=True))
        return ds_

    base = s * (ROWS // NT)
    npairs = (ROWS // NT) // SB1 // 2

    def body(b2, _):
        o0 = base + (2 * b2) * SB1
        dsA = fire(o0, gA, cA, dA, rowsA, cvA, semgA)
        dsB = fire(o0 + SB1, gB, cB, dB, rowsB, cvB, semgB)
        for d in dsA:
            d.wait()

        @pl.when(c == 0)
        def _():
            pltpu.sync_copy(cvA, sce_hbm.at[pl.ds(o0, SB1)])
        scale(rowsA, cvA)
        scA = scat(rowsA, dA, semsA)
        for d in dsB:
            d.wait()

        @pl.when(c == 0)
        def _():
            pltpu.sync_copy(cvB, sce_hbm.at[pl.ds(o0 + SB1, SB1)])
        scale(rowsB, cvB)
        scB = scat(rowsB, dB, semsB)
        for d in scA:
            d.wait()
        for d in scB:
            d.wait()
        return 0
    lax.fori_loop(0, npairs, body, 0)

    plsc.subcore_barrier()

    def fin(i, _):
        o = s * nr + i * BR
        pltpu.sync_copy(acc_sh.at[pl.ds(o, BR)], rowsA.at[pl.ds(0, BR)])
        pltpu.sync_copy(rowsA.at[pl.ds(0, BR)], out_hbm.at[c, pl.ds(o, BR)])
        return 0
    lax.fori_loop(0, nr // BR, fin, 0)


def _l2_body(gidx_hbm, dst_hbm, ytab_hbm, sce_hbm, zpad_hbm,
             out_hbm, acc_sh, gA, dA, gB, dB, ybA, ybB,
             cvA, cvB, zb, semgA, semgB, semsA, semsB):
    c = lax.axis_index("c")
    s = lax.axis_index("s")
    nz = NPAD // NT

    @pl.when(c == 0)
    def _():
        pltpu.sync_copy(zpad_hbm.at[pl.ds(s * nz, nz)], zb)

    @pl.when(c == 1)
    def _():
        def zf(i, _):
            zb[pl.ds(i * 16, 16)] = jnp.zeros((16,), jnp.float32)
            return 0
        lax.fori_loop(0, nz // 16, zf, 0)

    pltpu.sync_copy(zb, acc_sh.at[pl.ds(s * nz, nz)])
    plsc.subcore_barrier()

    def fire(o, g, dd, yb, cv, semg):
        pltpu.sync_copy(gidx_hbm.at[pl.ds(o, SB)], g)
        pltpu.sync_copy(dst_hbm.at[pl.ds(o, SB)], dd)
        pltpu.sync_copy(sce_hbm.at[pl.ds(o, SB)], cv)
        ds_ = []
        for k in range(SB):
            ds_.append(pltpu.async_copy(
                ytab_hbm.at[g.at[k]], yb.at[pl.ds(k * CHUNK, CHUNK)], semg))
        return ds_

    def scale(yb, cv):
        def mv(k, _):
            yb[pl.ds(k * 16, 16)] = (yb[pl.ds(k * 16, 16)]
                                     * cv[k // 8, pl.ds((k % 8) * 16, 16)])
            return 0
        lax.fori_loop(0, SB * CHUNK // 16, mv, 0)

    def scat(yb, dd, sems):
        ds_ = []
        for k in range(SB):
            ds_.append(pltpu.async_copy(
                yb.at[pl.ds(k * CHUNK, CHUNK)],
                acc_sh.at[dd.at[k]], sems, add=True))
        return ds_

    base = (c * NT + s) * (ROWS // (2 * NT))
    nblk = (ROWS // (2 * NT)) // SB

    def body(b2, _):
        o0 = base + (2 * b2) * SB
        dsA = fire(o0, gA, dA, ybA, cvA, semgA)
        dsB = fire(o0 + SB, gB, dB, ybB, cvB, semgB)
        for d in dsA:
            d.wait()
        scale(ybA, cvA)
        scA = scat(ybA, dA, semsA)
        for d in dsB:
            d.wait()
        scale(ybB, cvB)
        scB = scat(ybB, dB, semsB)
        for d in scA:
            d.wait()
        for d in scB:
            d.wait()
        return 0
    lax.fori_loop(0, nblk // 2, body, 0)

    oL = base + (nblk - 1) * SB
    dsA = fire(oL, gA, dA, ybA, cvA, semgA)
    for d in dsA:
        d.wait()
    scale(ybA, cvA)
    for d in scat(ybA, dA, semsA):
        d.wait()

    plsc.subcore_barrier()
    pltpu.sync_copy(acc_sh.at[pl.ds(s * nz, nz)], zb)
    pltpu.sync_copy(zb, out_hbm.at[c, pl.ds(s * nz, nz)])


_counts = pl.kernel(
    _counts_body,
    out_type=jax.ShapeDtypeStruct((2, RN_PAD), jnp.float32),
    mesh=_MESH,
    compiler_params=pltpu.CompilerParams(use_tc_tiling_on_sc=False),
    scratch_types=[
        pltpu.VMEM_SHARED((RN_PAD,), jnp.float32),
        pltpu.VMEM((KB, CHUNK), jnp.int32),
        pltpu.VMEM((CHUNK,), jnp.float32),
        pltpu.VMEM((RN_PAD // NT,), jnp.float32),
        pltpu.SemaphoreType.DMA,
    ],
)

_l1 = pl.kernel(
    _l1_body,
    out_type=[jax.ShapeDtypeStruct((2, NPAD, HH), jnp.float32),
              jax.ShapeDtypeStruct((ROWS, CHUNK), jnp.float32)],
    mesh=_MESH,
    compiler_params=pltpu.CompilerParams(use_tc_tiling_on_sc=False),
    scratch_types=[
        pltpu.VMEM_SHARED((NPAD, HH), jnp.float32),
        pltpu.VMEM((SB1, CHUNK), jnp.int32),
        pltpu.VMEM((SB1, CHUNK), jnp.int32),
        pltpu.VMEM((SB1, CHUNK), jnp.int32),
        pltpu.VMEM((SB1, CHUNK), jnp.int32),
        pltpu.VMEM((SB1, CHUNK), jnp.int32),
        pltpu.VMEM((SB1, CHUNK), jnp.int32),
        pltpu.VMEM((SB1 * CHUNK, HH), jnp.float32),
        pltpu.VMEM((SB1 * CHUNK, HH), jnp.float32),
        pltpu.VMEM((SB1, CHUNK), jnp.float32),
        pltpu.VMEM((SB1, CHUNK), jnp.float32),
        pltpu.SemaphoreType.DMA,
        pltpu.SemaphoreType.DMA,
        pltpu.SemaphoreType.DMA,
        pltpu.SemaphoreType.DMA,
    ],
)

_l2 = pl.kernel(
    _l2_body,
    out_type=jax.ShapeDtypeStruct((2, NPAD), jnp.float32),
    mesh=_MESH,
    compiler_params=pltpu.CompilerParams(use_tc_tiling_on_sc=False),
    scratch_types=[
        pltpu.VMEM_SHARED((NPAD,), jnp.float32),
        pltpu.VMEM((SB, CHUNK), jnp.int32),
        pltpu.VMEM((SB, CHUNK), jnp.int32),
        pltpu.VMEM((SB, CHUNK), jnp.int32),
        pltpu.VMEM((SB, CHUNK), jnp.int32),
        pltpu.VMEM((SB * CHUNK,), jnp.float32),
        pltpu.VMEM((SB * CHUNK,), jnp.float32),
        pltpu.VMEM((SB, CHUNK), jnp.float32),
        pltpu.VMEM((SB, CHUNK), jnp.float32),
        pltpu.VMEM((NPAD // NT,), jnp.float32),
        pltpu.SemaphoreType.DMA,
        pltpu.SemaphoreType.DMA,
        pltpu.SemaphoreType.DMA,
        pltpu.SemaphoreType.DMA,
    ],
)


@jax.jit
def kernel(x_user, x_item, edge_index, edge_type, emb_user, emb_item,
           comp1, bases1, root1, bias1, comp2, bases2, root2, bias2):
    x = jnp.concatenate([jnp.take(emb_user, x_user, axis=0),
                         jnp.take(emb_item, x_item, axis=0)], axis=0)
    src = edge_index[0]
    dst = edge_index[1]
    et = edge_type
    npe = E_PAD - E_EDGES
    gidx = jnp.concatenate(
        [et * N_NODES + src, jnp.zeros((npe,), jnp.int32)]).reshape(ROWS, CHUNK)
    cidx = jnp.concatenate(
        [et * N_NODES + dst, jnp.full((npe,), RN, jnp.int32)]).reshape(ROWS, CHUNK)
    dstp = jnp.concatenate(
        [dst, jnp.zeros((npe,), jnp.int32)]).reshape(ROWS, CHUNK)

    w1flat, w2ct = pl.pallas_call(
        _w_kernel,
        out_shape=[jax.ShapeDtypeStruct((R, H * H), jnp.float32),
                   jax.ShapeDtypeStruct((H, R), jnp.float32)],
    )(comp1, bases1.reshape(NB_BASES, H * H),
      bases2[:, :, 1].T, comp2.T)
    w1h = w1flat.reshape(R, H, 2, HH).transpose(2, 0, 1, 3)
    root1h = root1.reshape(H, 2, HH).transpose(1, 0, 2)
    bias1h = bias1.reshape(2, 1, HH)

    nblk = N_NODES // BN1
    tab, rootp = pl.pallas_call(
        _dense1_kernel,
        grid=(2, nblk),
        in_specs=[
            pl.BlockSpec((BN1, H), lambda h, i: (i, 0)),
            pl.BlockSpec((1, R, H, HH), lambda h, i: (h, 0, 0, 0)),
            pl.BlockSpec((1, H, HH), lambda h, i: (h, 0, 0)),
            pl.BlockSpec((1, 1, HH), lambda h, i: (h, 0, 0)),
        ],
        out_specs=[
            pl.BlockSpec((1, R, BN1, HH), lambda h, i: (h, 0, i, 0)),
            pl.BlockSpec((1, BN1, HH), lambda h, i: (h, i, 0)),
        ],
        out_shape=[jax.ShapeDtypeStruct((2, R, N_NODES, HH), jnp.float32),
                   jax.ShapeDtypeStruct((2, N_NODES, HH), jnp.float32)],
    )(x, w1h, root1h, bias1h)
    tab = tab.reshape(2, RN, HH)

    parts = _counts(cidx)
    invc = pl.pallas_call(
        _invc_kernel,
        out_shape=jax.ShapeDtypeStruct((RN_PAD // 128, 128), jnp.float32),
    )(parts.reshape(2, RN_PAD // 128, 128)).reshape(RN_PAD)

    rootp_pad = jnp.pad(rootp, ((0, 0), (0, NPAD - N_NODES), (0, 0)))
    x1parts, sce = _l1(gidx, cidx, dstp, tab, invc, rootp_pad)

    w2all = jnp.concatenate(
        [w2ct, root2[:, 1:2], jnp.zeros((H, 3), jnp.float32)], axis=1)
    bvec = jnp.concatenate(
        [jnp.zeros((4,), jnp.float32), bias2[1:2],
         jnp.zeros((3,), jnp.float32)]).reshape(1, 8)
    bn2 = NPAD // NT
    d2 = pl.pallas_call(
        _dense2_kernel,
        grid=(NT,),
        in_specs=[
            pl.BlockSpec((1, bn2, HH), lambda i: (0, i, 0)),
            pl.BlockSpec((1, bn2, HH), lambda i: (1, i, 0)),
            pl.BlockSpec((HH, 8), lambda i: (0, 0)),
            pl.BlockSpec((HH, 8), lambda i: (1, 0)),
            pl.BlockSpec((1, 8), lambda i: (0, 0)),
        ],
        out_specs=pl.BlockSpec((bn2, 8), lambda i: (i, 0)),
        out_shape=jax.ShapeDtypeStruct((NPAD, 8), jnp.float32),
    )(x1parts, x1parts, w2all, w2all, bvec)

    ytab = d2[:N_NODES, :R].T.reshape(RN)
    zpad = d2[:, R]

    l2parts = _l2(gidx, dstp, ytab, sce, zpad)

    out = pl.pallas_call(
        _fin_kernel,
        out_shape=jax.ShapeDtypeStruct((NPAD // 128, 128), jnp.float32),
    )(l2parts.reshape(2, NPAD // 128, 128)).reshape(NPAD)
    return out[:N_NODES]


# speedup vs baseline: 1.0149x; 1.0149x over previous
import jax
import jax.numpy as jnp
from jax import lax
from jax.experimental import pallas as pl
from jax.experimental.pallas import tpu as pltpu
from jax.experimental.pallas import tpu_sc as plsc

N_NODES = 50000
E_EDGES = 800000
H = 64
HH = 32
R = 4
NB_BASES = 30

CHUNK = 128
ROWS = 6400
E_PAD = ROWS * CHUNK
RN = R * N_NODES
RN_PAD = 200704
NPAD = 51200
KB = 40
NT = 16
SB = 8
SB1 = 2


def _w_kernel(comp1_ref, b1flat_ref, b2ct_ref, comp2t_ref, w1_ref, w2_ref):
    w1_ref[...] = jnp.dot(comp1_ref[...], b1flat_ref[...],
                          preferred_element_type=jnp.float32)
    w2_ref[...] = jnp.dot(b2ct_ref[...], comp2t_ref[...],
                          preferred_element_type=jnp.float32)


BN1 = 2000


def _dense1_kernel(x_ref, w1_ref, root_ref, bias_ref, tab_ref, rp_ref):
    xb = x_ref[...]
    for r in range(R):
        tab_ref[0, r] = jnp.dot(xb, w1_ref[0, r],
                                preferred_element_type=jnp.float32)
    rp_ref[0] = jnp.dot(xb, root_ref[0],
                        preferred_element_type=jnp.float32) + bias_ref[0]


def _invc_kernel(parts_ref, invc_ref):
    p = parts_ref[0] + parts_ref[1]
    inv = 1.0 / jnp.maximum(p, 1.0)
    nrows = RN_PAD // 128
    g = (lax.broadcasted_iota(jnp.int32, (nrows, 128), 0) * 128
         + lax.broadcasted_iota(jnp.int32, (nrows, 128), 1))
    invc_ref[...] = jnp.where(g < RN, inv, 0.0)


def _dense2_kernel(a0_ref, a1_ref, wa_ref, wb_ref, bias_ref, out_ref):
    out_ref[...] = (
        jnp.dot(a0_ref[0], wa_ref[...], preferred_element_type=jnp.float32)
        + jnp.dot(a1_ref[0], wb_ref[...], preferred_element_type=jnp.float32)
        + bias_ref[...])


def _fin_kernel(a_ref, o_ref):
    o_ref[...] = a_ref[0] + a_ref[1]


_MESH = plsc.VectorSubcoreMesh(core_axis_name="c", subcore_axis_name="s")


def _counts_body(cidx_hbm, out_hbm, acc_sh, cbuf, ones, zbuf, sem):
    c = lax.axis_index("c")
    s = lax.axis_index("s")
    nz = RN_PAD // NT

    def zf(i, _):
        zbuf[pl.ds(i * 16, 16)] = jnp.zeros((16,), jnp.float32)
        return 0
    lax.fori_loop(0, nz // 16, zf, 0)
    pltpu.sync_copy(zbuf, acc_sh.at[pl.ds(s * nz, nz)])

    def of(i, _):
        ones[pl.ds(i * 16, 16)] = jnp.ones((16,), jnp.float32)
        return 0
    lax.fori_loop(0, CHUNK // 16, of, 0)
    plsc.subcore_barrier()

    base = (c * NT + s) * (ROWS // (2 * NT))

    def blk(b, _):
        pltpu.sync_copy(cidx_hbm.at[pl.ds(base + b * KB, KB)], cbuf)

        def ch(j, _):
            pltpu.sync_copy(ones, acc_sh.at[cbuf.at[j]], add=True)
            return 0
        lax.fori_loop(0, KB, ch, 0)
        return 0
    lax.fori_loop(0, (ROWS // (2 * NT)) // KB, blk, 0)

    plsc.subcore_barrier()
    pltpu.sync_copy(acc_sh.at[pl.ds(s * nz, nz)], zbuf)
    pltpu.sync_copy(zbuf, out_hbm.at[c, pl.ds(s * nz, nz)])


def _l1_body(gidx_hbm, cidx_hbm, dst_hbm, tab_hbm, invc_hbm, rootp_hbm,
             out_hbm, acc_sh, gA, cA, dA, gB, cB, dB, rowsA, rowsB,
             cvA, cvB, semgA, semgB, semsA, semsB):
    c = lax.axis_index("c")
    s = lax.axis_index("s")
    nr = NPAD // NT
    BR = 200

    def ini(i, _):
        o = s * nr + i * BR
        pltpu.sync_copy(rootp_hbm.at[c, pl.ds(o, BR)],
                        rowsA.at[pl.ds(0, BR)])
        pltpu.sync_copy(rowsA.at[pl.ds(0, BR)], acc_sh.at[pl.ds(o, BR)])
        return 0
    lax.fori_loop(0, nr // BR, ini, 0)
    plsc.subcore_barrier()

    def fire(o, g, cc, dd, rows, cv, semg):
        pltpu.sync_copy(gidx_hbm.at[pl.ds(o, SB1)], g)
        pltpu.sync_copy(cidx_hbm.at[pl.ds(o, SB1)], cc)
        pltpu.sync_copy(dst_hbm.at[pl.ds(o, SB1)], dd)
        ds_ = []
        for k in range(SB1):
            ds_.append(pltpu.async_copy(
                tab_hbm.at[c].at[g.at[k]],
                rows.at[pl.ds(k * CHUNK, CHUNK)], semg))
            ds_.append(pltpu.async_copy(
                invc_hbm.at[cc.at[k]],
                cv.at[pl.ds(k * CHUNK, CHUNK)], semg))
        return ds_

    def scale(rows, cv):
        def sck(k, _):
            sv16 = cv[pl.ds(k * 16, 16)]
            for e16 in range(16):
                e = k * 16 + e16
                sv = sv16[e16]
                rows[e, pl.ds(0, 16)] = rows[e, pl.ds(0, 16)] * sv
                rows[e, pl.ds(16, 16)] = rows[e, pl.ds(16, 16)] * sv
            return 0
        lax.fori_loop(0, SB1 * CHUNK // 16, sck, 0)

    def scat(rows, dd, sems):
        ds_ = []
        for k in range(SB1):
            ds_.append(pltpu.async_copy(
                rows.at[pl.ds(k * CHUNK, CHUNK)],
                acc_sh.at[dd.at[k]], sems, add=True))
        return ds_

    base = s * (ROWS // NT)
    npairs = (ROWS // NT) // SB1 // 2

    def body(b2, _):
        o0 = base + (2 * b2) * SB1
        dsA = fire(o0, gA, cA, dA, rowsA, cvA, semgA)
        dsB = fire(o0 + SB1, gB, cB, dB, rowsB, cvB, semgB)
        for d in dsA:
            d.wait()
        scale(rowsA, cvA)
        scA = scat(rowsA, dA, semsA)
        for d in dsB:
            d.wait()
        scale(rowsB, cvB)
        scB = scat(rowsB, dB, semsB)
        for d in scA:
            d.wait()
        for d in scB:
            d.wait()
        return 0
    lax.fori_loop(0, npairs, body, 0)

    plsc.subcore_barrier()

    def fin(i, _):
        o = s * nr + i * BR
        pltpu.sync_copy(acc_sh.at[pl.ds(o, BR)], rowsA.at[pl.ds(0, BR)])
        pltpu.sync_copy(rowsA.at[pl.ds(0, BR)], out_hbm.at[c, pl.ds(o, BR)])
        return 0
    lax.fori_loop(0, nr // BR, fin, 0)


def _l2_body(gidx_hbm, cidx_hbm, dst_hbm, ytab_hbm, invc_hbm, zpad_hbm,
             out_hbm, acc_sh, gA, cA, dA, gB, cB, dB, ybA, ybB,
             cvA, cvB, zb, semgA, semgB, semsA, semsB):
    c = lax.axis_index("c")
    s = lax.axis_index("s")
    nz = NPAD // NT

    @pl.when(c == 0)
    def _():
        pltpu.sync_copy(zpad_hbm.at[pl.ds(s * nz, nz)], zb)

    @pl.when(c == 1)
    def _():
        def zf(i, _):
            zb[pl.ds(i * 16, 16)] = jnp.zeros((16,), jnp.float32)
            return 0
        lax.fori_loop(0, nz // 16, zf, 0)

    pltpu.sync_copy(zb, acc_sh.at[pl.ds(s * nz, nz)])
    plsc.subcore_barrier()

    def fire(o, g, cc, dd, yb, cv, semg):
        pltpu.sync_copy(gidx_hbm.at[pl.ds(o, SB)], g)
        pltpu.sync_copy(cidx_hbm.at[pl.ds(o, SB)], cc)
        pltpu.sync_copy(dst_hbm.at[pl.ds(o, SB)], dd)
        ds_ = []
        for k in range(SB):
            ds_.append(pltpu.async_copy(
                ytab_hbm.at[g.at[k]], yb.at[pl.ds(k * CHUNK, CHUNK)], semg))
            ds_.append(pltpu.async_copy(
                invc_hbm.at[cc.at[k]], cv.at[pl.ds(k * CHUNK, CHUNK)], semg))
        return ds_

    def scale(yb, cv):
        def mv(k, _):
            yb[pl.ds(k * 16, 16)] = (yb[pl.ds(k * 16, 16)]
                                     * cv[pl.ds(k * 16, 16)])
            return 0
        lax.fori_loop(0, SB * CHUNK // 16, mv, 0)

    def scat(yb, dd, sems):
        ds_ = []
        for k in range(SB):
            ds_.append(pltpu.async_copy(
                yb.at[pl.ds(k * CHUNK, CHUNK)],
                acc_sh.at[dd.at[k]], sems, add=True))
        return ds_

    base = (c * NT + s) * (ROWS // (2 * NT))
    nblk = (ROWS // (2 * NT)) // SB

    def body(b2, _):
        o0 = base + (2 * b2) * SB
        dsA = fire(o0, gA, cA, dA, ybA, cvA, semgA)
        dsB = fire(o0 + SB, gB, cB, dB, ybB, cvB, semgB)
        for d in dsA:
            d.wait()
        scale(ybA, cvA)
        scA = scat(ybA, dA, semsA)
        for d in dsB:
            d.wait()
        scale(ybB, cvB)
        scB = scat(ybB, dB, semsB)
        for d in scA:
            d.wait()
        for d in scB:
            d.wait()
        return 0
    lax.fori_loop(0, nblk // 2, body, 0)

    oL = base + (nblk - 1) * SB
    dsA = fire(oL, gA, cA, dA, ybA, cvA, semgA)
    for d in dsA:
        d.wait()
    scale(ybA, cvA)
    for d in scat(ybA, dA, semsA):
        d.wait()

    plsc.subcore_barrier()
    pltpu.sync_copy(acc_sh.at[pl.ds(s * nz, nz)], zb)
    pltpu.sync_copy(zb, out_hbm.at[c, pl.ds(s * nz, nz)])


_counts = pl.kernel(
    _counts_body,
    out_type=jax.ShapeDtypeStruct((2, RN_PAD), jnp.float32),
    mesh=_MESH,
    compiler_params=pltpu.CompilerParams(use_tc_tiling_on_sc=False),
    scratch_types=[
        pltpu.VMEM_SHARED((RN_PAD,), jnp.float32),
        pltpu.VMEM((KB, CHUNK), jnp.int32),
        pltpu.VMEM((CHUNK,), jnp.float32),
        pltpu.VMEM((RN_PAD // NT,), jnp.float32),
        pltpu.SemaphoreType.DMA,
    ],
)

_l1 = pl.kernel(
    _l1_body,
    out_type=jax.ShapeDtypeStruct((2, NPAD, HH), jnp.float32),
    mesh=_MESH,
    compiler_params=pltpu.CompilerParams(use_tc_tiling_on_sc=False),
    scratch_types=[
        pltpu.VMEM_SHARED((NPAD, HH), jnp.float32),
        pltpu.VMEM((SB1, CHUNK), jnp.int32),
        pltpu.VMEM((SB1, CHUNK), jnp.int32),
        pltpu.VMEM((SB1, CHUNK), jnp.int32),
        pltpu.VMEM((SB1, CHUNK), jnp.int32),
        pltpu.VMEM((SB1, CHUNK), jnp.int32),
        pltpu.VMEM((SB1, CHUNK), jnp.int32),
        pltpu.VMEM((SB1 * CHUNK, HH), jnp.float32),
        pltpu.VMEM((SB1 * CHUNK, HH), jnp.float32),
        pltpu.VMEM((SB1 * CHUNK,), jnp.float32),
        pltpu.VMEM((SB1 * CHUNK,), jnp.float32),
        pltpu.SemaphoreType.DMA,
        pltpu.SemaphoreType.DMA,
        pltpu.SemaphoreType.DMA,
        pltpu.SemaphoreType.DMA,
    ],
)

_l2 = pl.kernel(
    _l2_body,
    out_type=jax.ShapeDtypeStruct((2, NPAD), jnp.float32),
    mesh=_MESH,
    compiler_params=pltpu.CompilerParams(use_tc_tiling_on_sc=False),
    scratch_types=[
        pltpu.VMEM_SHARED((NPAD,), jnp.float32),
        pltpu.VMEM((SB, CHUNK), jnp.int32),
        pltpu.VMEM((SB, CHUNK), jnp.int32),
        pltpu.VMEM((SB, CHUNK), jnp.int32),
        pltpu.VMEM((SB, CHUNK), jnp.int32),
        pltpu.VMEM((SB, CHUNK), jnp.int32),
        pltpu.VMEM((SB, CHUNK), jnp.int32),
        pltpu.VMEM((SB * CHUNK,), jnp.float32),
        pltpu.VMEM((SB * CHUNK,), jnp.float32),
        pltpu.VMEM((SB * CHUNK,), jnp.float32),
        pltpu.VMEM((SB * CHUNK,), jnp.float32),
        pltpu.VMEM((NPAD // NT,), jnp.float32),
        pltpu.SemaphoreType.DMA,
        pltpu.SemaphoreType.DMA,
        pltpu.SemaphoreType.DMA,
        pltpu.SemaphoreType.DMA,
    ],
)


@jax.jit
def kernel(x_user, x_item, edge_index, edge_type, emb_user, emb_item,
           comp1, bases1, root1, bias1, comp2, bases2, root2, bias2):
    x = jnp.concatenate([jnp.take(emb_user, x_user, axis=0),
                         jnp.take(emb_item, x_item, axis=0)], axis=0)
    src = edge_index[0]
    dst = edge_index[1]
    et = edge_type
    npe = E_PAD - E_EDGES
    gidx = jnp.concatenate(
        [et * N_NODES + src, jnp.zeros((npe,), jnp.int32)]).reshape(ROWS, CHUNK)
    cidx = jnp.concatenate(
        [et * N_NODES + dst, jnp.full((npe,), RN, jnp.int32)]).reshape(ROWS, CHUNK)
    dstp = jnp.concatenate(
        [dst, jnp.zeros((npe,), jnp.int32)]).reshape(ROWS, CHUNK)

    w1flat, w2ct = pl.pallas_call(
        _w_kernel,
        out_shape=[jax.ShapeDtypeStruct((R, H * H), jnp.float32),
                   jax.ShapeDtypeStruct((H, R), jnp.float32)],
    )(comp1, bases1.reshape(NB_BASES, H * H),
      bases2[:, :, 1].T, comp2.T)
    w1h = w1flat.reshape(R, H, 2, HH).transpose(2, 0, 1, 3)
    root1h = root1.reshape(H, 2, HH).transpose(1, 0, 2)
    bias1h = bias1.reshape(2, 1, HH)

    nblk = N_NODES // BN1
    tab, rootp = pl.pallas_call(
        _dense1_kernel,
        grid=(2, nblk),
        in_specs=[
            pl.BlockSpec((BN1, H), lambda h, i: (i, 0)),
            pl.BlockSpec((1, R, H, HH), lambda h, i: (h, 0, 0, 0)),
            pl.BlockSpec((1, H, HH), lambda h, i: (h, 0, 0)),
            pl.BlockSpec((1, 1, HH), lambda h, i: (h, 0, 0)),
        ],
        out_specs=[
            pl.BlockSpec((1, R, BN1, HH), lambda h, i: (h, 0, i, 0)),
            pl.BlockSpec((1, BN1, HH), lambda h, i: (h, i, 0)),
        ],
        out_shape=[jax.ShapeDtypeStruct((2, R, N_NODES, HH), jnp.float32),
                   jax.ShapeDtypeStruct((2, N_NODES, HH), jnp.float32)],
    )(x, w1h, root1h, bias1h)
    tab = tab.reshape(2, RN, HH)

    parts = _counts(cidx)
    invc = pl.pallas_call(
        _invc_kernel,
        out_shape=jax.ShapeDtypeStruct((RN_PAD // 128, 128), jnp.float32),
    )(parts.reshape(2, RN_PAD // 128, 128)).reshape(RN_PAD)

    rootp_pad = jnp.pad(rootp, ((0, 0), (0, NPAD - N_NODES), (0, 0)))
    x1parts = _l1(gidx, cidx, dstp, tab, invc, rootp_pad)

    w2all = jnp.concatenate(
        [w2ct, root2[:, 1:2], jnp.zeros((H, 3), jnp.float32)], axis=1)
    bvec = jnp.concatenate(
        [jnp.zeros((4,), jnp.float32), bias2[1:2],
         jnp.zeros((3,), jnp.float32)]).reshape(1, 8)
    bn2 = NPAD // NT
    d2 = pl.pallas_call(
        _dense2_kernel,
        grid=(NT,),
        in_specs=[
            pl.BlockSpec((1, bn2, HH), lambda i: (0, i, 0)),
            pl.BlockSpec((1, bn2, HH), lambda i: (1, i, 0)),
            pl.BlockSpec((HH, 8), lambda i: (0, 0)),
            pl.BlockSpec((HH, 8), lambda i: (1, 0)),
            pl.BlockSpec((1, 8), lambda i: (0, 0)),
        ],
        out_specs=pl.BlockSpec((bn2, 8), lambda i: (i, 0)),
        out_shape=jax.ShapeDtypeStruct((NPAD, 8), jnp.float32),
    )(x1parts, x1parts, w2all, w2all, bvec)

    ytab = d2[:N_NODES, :R].T.reshape(RN)
    zpad = d2[:, R]

    l2parts = _l2(gidx, cidx, dstp, ytab, invc, zpad)

    out = pl.pallas_call(
        _fin_kernel,
        out_shape=jax.ShapeDtypeStruct((NPAD // 128, 128), jnp.float32),
    )(l2parts.reshape(2, NPAD // 128, 128)).reshape(NPAD)
    return out[:N_NODES]
